# baseline (speedup 1.0000x reference)
import functools

import numpy as np
import jax
import jax.numpy as jnp
from jax import lax
from jax.experimental import pallas as pl
from jax.experimental.pallas import tpu as pltpu
from jax.experimental.pallas import tpu_sc as plsc

_VOCAB = 100000
_N_TABLES = 26
_EMBED = 32
_B = 4096
_N_FEAT = 1 + _N_TABLES
_DI_DIM = _N_FEAT * (_N_FEAT + 1) // 2

_NC, _NS = 2, 16
_NW = _NC * _NS
_TOTAL = _B * _N_TABLES
_BPW = _TOTAL // _NW
_CHUNK = 128
_NCHUNK = _BPW // _CHUNK

_ROWS = _N_TABLES * _VOCAB
_RM_ROWS = _ROWS // 4
_TCOLS = (_ROWS + 127) // 128
_FULL_TCOLS = _ROWS // 128
_TAIL_W = _ROWS - _FULL_TCOLS * 128
_KMAX = 636


def _sc_transpose(tableT, tail_rm):
    mesh = plsc.VectorSubcoreMesh(core_axis_name="c", subcore_axis_name="s")

    @functools.partial(
        pl.kernel,
        mesh=mesh,
        out_type=jax.ShapeDtypeStruct((_RM_ROWS, 128), jnp.float32),
        scratch_types=[
            pltpu.VMEM((32, 128), jnp.float32),
            pltpu.VMEM((32, 128), jnp.float32),
            pltpu.VMEM((32, 128), jnp.float32),
            pltpu.VMEM((32, 128), jnp.float32),
            pltpu.VMEM((128, 16), jnp.int32),
            pltpu.VMEM((128, 16), jnp.int32),
            pltpu.VMEM((128, 16), jnp.int32),
            pltpu.VMEM((128, 16), jnp.int32),
            pltpu.SemaphoreType.DMA,
            pltpu.SemaphoreType.DMA,
        ],
        compiler_params=pltpu.CompilerParams(needs_layout_passes=False),
    )
    def k(tableT_hbm, tail_hbm, out_hbm, slab0, slab1, outb0, outb1,
          tv_r, pv_r, q0_r, q1_r, isem, osem):
        wid = lax.axis_index("s") * _NC + lax.axis_index("c")
        e16 = lax.iota(jnp.int32, 16)

        def setup(j, carry):
            t = ((e16 + (j & 15)) & 15) + (j & ~jnp.int32(15))
            q = (t & 3) << 5
            tv_r[j] = t
            pv_r[j] = t >> 2
            q0_r[j] = q + e16
            q1_r[j] = q + e16 + 16
            return carry

        lax.fori_loop(0, 128, setup, 0)

        def col(kk):
            return wid + _NW * kk

        def fire_in(kk, slab):
            pltpu.async_copy(
                tableT_hbm.at[:, pl.ds(col(kk) * 128, 128)], slab, isem)

        def drain(sem, buf):
            pltpu.make_async_copy(tableT_hbm.at[:, pl.ds(0, 128)], buf, sem).wait()

        e16b = e16 + 16

        def transpose(slab, outb):
            def tbody(j4, carry):
                for u in range(4):
                    j = j4 * 4 + u
                    tv = tv_r[j]
                    pv = pv_r[j]
                    lo = plsc.load_gather(slab, [e16, tv])
                    plsc.store_scatter(outb, [pv, q0_r[j]], lo)
                    hi = plsc.load_gather(slab, [e16b, tv])
                    plsc.store_scatter(outb, [pv, q1_r[j]], hi)
                return carry

            lax.fori_loop(0, 32, tbody, 0)

        def step(kk, slab, outb, first):
            nxt = slab1 if slab is slab0 else slab0

            @pl.when(col(kk + 1) < _FULL_TCOLS)
            def _():
                fire_in(kk + 1, nxt)

            @pl.when(col(kk) < _FULL_TCOLS)
            def _():
                drain(isem, slab)
                transpose(slab, outb)
                if not first:
                    drain(osem, outb)
                pltpu.async_copy(outb, out_hbm.at[pl.ds(col(kk) * 32, 32)], osem)

        fire_in(0, slab0)

        def body(k2, carry):
            step(2 * k2, slab0, outb0, first=False)
            step(2 * k2 + 1, slab1, outb1, first=False)
            return carry

        step(0, slab0, outb0, first=True)
        step(1, slab1, outb1, first=True)
        lax.fori_loop(1, _KMAX // 2, body, 0)
        drain(osem, outb0)
        drain(osem, outb1)

        @pl.when(wid == _FULL_TCOLS % _NW)
        def _():
            pltpu.sync_copy(tail_hbm, out_hbm.at[pl.ds(_FULL_TCOLS * 32, 16)])

    return k(tableT, tail_rm)


def _sc_gather(table_rm, idx3d):
    mesh = plsc.VectorSubcoreMesh(core_axis_name="c", subcore_axis_name="s")

    @functools.partial(
        pl.kernel,
        mesh=mesh,
        out_type=jax.ShapeDtypeStruct((_EMBED, _TOTAL), jnp.float32),
        scratch_types=[
            pltpu.VMEM((_NCHUNK, _CHUNK), jnp.int32),
            pltpu.VMEM((_NCHUNK, _CHUNK), jnp.int32),
            pltpu.VMEM((_CHUNK, 128), jnp.float32),
            pltpu.VMEM((_CHUNK, 128), jnp.float32),
            pltpu.VMEM((_EMBED, _CHUNK), jnp.float32),
            pltpu.SemaphoreType.DMA,
        ],
        compiler_params=pltpu.CompilerParams(needs_layout_passes=False),
    )
    def k(tbl_hbm, idx_hbm, out_hbm, idxr_v, idx4_v, g0, g1, outc_v, sem):
        wid = lax.axis_index("s") * _NC + lax.axis_index("c")
        e16 = lax.iota(jnp.int32, 16)
        pltpu.sync_copy(idx_hbm.at[wid], idxr_v)
        for c in range(_NCHUNK):
            for g in range(8):
                idx4_v[c, g * 16:(g + 1) * 16] = (
                    idxr_v[c, g * 16:(g + 1) * 16] >> 2)

        def fire(c, gbuf):
            pltpu.async_copy(tbl_hbm.at[idx4_v.at[c]], gbuf, sem)

        def drain(gbuf):
            pltpu.make_async_copy(tbl_hbm.at[pl.ds(0, _CHUNK)], gbuf, sem).wait()

        def extract(c, gbuf):
            for g in range(8):
                li = idxr_v[c, g * 16:(g + 1) * 16] & 3
                colbase = li * 32
                rowv = e16 + g * 16
                for e in range(_EMBED):
                    vals = plsc.load_gather(gbuf, [rowv, colbase + e])
                    outc_v[e, g * 16:(g + 1) * 16] = vals
            pltpu.sync_copy(
                outc_v,
                out_hbm.at[:, pl.ds(wid * _BPW + c * _CHUNK, _CHUNK)])

        fire(0, g0)

        def body(c2, carry):
            c = 2 * c2

            @pl.when(c + 1 < _NCHUNK)
            def _():
                fire(c + 1, g1)

            drain(g0)
            extract(c, g0)

            @pl.when(c + 2 < _NCHUNK)
            def _():
                fire(c + 2, g0)

            @pl.when(c + 1 < _NCHUNK)
            def _():
                drain(g1)
                extract(c + 1, g1)
            return carry

        lax.fori_loop(0, (_NCHUNK + 1) // 2, body, 0)

    return k(table_rm, idx3d)


_BLK = 256
_GRID = _B // _BLK


def _dense_body(dense_ref, embed_ref,
                bw0, bb0, bw1, bb1, bw2, bb2,
                tw0, tb0, tw1, tb1, tw2, tb2, tw3, tb3, tw4, tb4,
                out_ref, acc_ref):
    h = dense_ref[:]
    h = jnp.maximum(h @ bw0[:] + bb0[:], 0.0)
    h = jnp.maximum(h @ bw1[:] + bb1[:], 0.0)
    bot = jnp.maximum(h @ bw2[:] + bb2[:], 0.0)

    feat = jnp.concatenate([bot.reshape(_BLK, 1, _EMBED), embed_ref[:]], axis=1)
    acc_ref[:, 0:_EMBED] = bot
    off = _EMBED
    for i in range(_N_FEAT):
        gi = jnp.sum(feat * feat[:, i:i + 1, :], axis=2)
        w = _N_FEAT - i
        acc_ref[:, off:off + w] = gi[:, i:]
        off += w

    x = acc_ref[:]
    x = jnp.maximum(x @ tw0[:] + tb0[:], 0.0)
    x = jnp.maximum(x @ tw1[:] + tb1[:], 0.0)
    x = jnp.maximum(x @ tw2[:] + tb2[:], 0.0)
    x = jnp.maximum(x @ tw3[:] + tb3[:], 0.0)
    x = x @ tw4[:] + tb4[:]
    out_ref[:] = jax.nn.sigmoid(x)


def _dense_call(dense, embed, bw0, bb0, bw1, bb1, bw2, bb2,
                tw0, tb0, tw1, tb1, tw2, tb2, tw3, tb3, tw4, tb4):
    def full(a):
        return pl.BlockSpec(a.shape, lambda i: (0,) * a.ndim)

    ws = (bw0, bb0, bw1, bb1, bw2, bb2,
          tw0, tb0, tw1, tb1, tw2, tb2, tw3, tb3, tw4, tb4)
    return pl.pallas_call(
        _dense_body,
        grid=(_GRID,),
        in_specs=[
            pl.BlockSpec((_BLK, dense.shape[1]), lambda i: (i, 0)),
            pl.BlockSpec((_BLK, _N_TABLES, _EMBED), lambda i: (i, 0, 0)),
        ] + [full(w) for w in ws],
        out_specs=pl.BlockSpec((_BLK, 1), lambda i: (i, 0)),
        out_shape=jax.ShapeDtypeStruct((_B, 1), jnp.float32),
        scratch_shapes=[pltpu.VMEM((_BLK, _EMBED + _DI_DIM), jnp.float32)],
        compiler_params=pltpu.CompilerParams(
            dimension_semantics=("arbitrary",),
        ),
    )(dense, embed, *ws)


def kernel(dense_features, cat_features, embedding_table,
           bw0, bb0, bw1, bb1, bw2, bb2,
           tw0, tb0, tw1, tb1, tw2, tb2, tw3, tb3, tw4, tb4):
    offsets = jnp.asarray(np.arange(_N_TABLES, dtype=np.int32) * _VOCAB)
    idx = (cat_features + offsets[None, :]).reshape(_NW, _NCHUNK, _CHUNK)
    tail_rm = embedding_table[_FULL_TCOLS * 128:].reshape(16, 128)
    table_rm = _sc_transpose(embedding_table.T, tail_rm)
    rowsT = _sc_gather(table_rm, idx)
    embed = rowsT.T.reshape(_B, _N_TABLES, _EMBED)
    b2 = lambda v: v.reshape(1, -1)
    return _dense_call(dense_features, embed,
                       bw0, b2(bb0), bw1, b2(bb1), bw2, b2(bb2),
                       tw0, b2(tb0), tw1, b2(tb1), tw2, b2(tb2),
                       tw3, b2(tb3), tw4, b2(tb4))


# speedup vs baseline: 1.4479x; 1.4479x over previous
import functools

import numpy as np
import jax
import jax.numpy as jnp
from jax import lax
from jax.experimental import pallas as pl
from jax.experimental.pallas import tpu as pltpu
from jax.experimental.pallas import tpu_sc as plsc

_VOCAB = 100000
_N_TABLES = 26
_EMBED = 32
_B = 4096
_N_FEAT = 1 + _N_TABLES
_DI_DIM = _N_FEAT * (_N_FEAT + 1) // 2


_NC, _NS = 2, 16
_NW = _NC * _NS
_TOTAL = _B * _N_TABLES
_BPW = _TOTAL // _NW
_CHUNK = 128
_NCHUNK = _BPW // _CHUNK


def _sc_gather(table, idx3d):
    mesh = plsc.VectorSubcoreMesh(core_axis_name="c", subcore_axis_name="s")

    @functools.partial(
        pl.kernel,
        mesh=mesh,
        out_type=jax.ShapeDtypeStruct((_TOTAL, _EMBED), jnp.float32),
        scratch_types=[
            pltpu.VMEM((_NCHUNK, _CHUNK), jnp.int32),
            pltpu.VMEM((_BPW, _EMBED), jnp.float32),
            pltpu.SemaphoreType.DMA,
        ],
        compiler_params=pltpu.CompilerParams(use_tc_tiling_on_sc=False),
    )
    def k(table_hbm, idx_hbm, out_hbm, idx_v, rows_v, sem):
        wid = lax.axis_index("s") * _NC + lax.axis_index("c")
        base = wid * _BPW
        pltpu.sync_copy(idx_hbm.at[wid], idx_v)

        def body(c, carry):
            pltpu.async_copy(
                table_hbm.at[idx_v.at[c]],
                rows_v.at[pl.ds(c * _CHUNK, _CHUNK)],
                sem,
            ).wait()
            return carry

        lax.fori_loop(0, _NCHUNK, body, 0)
        pltpu.sync_copy(rows_v, out_hbm.at[pl.ds(base, _BPW)])

    return k(table, idx3d)


_BLK = 256
_GRID = _B // _BLK


def _dense_body(dense_ref, embed_ref,
                bw0, bb0, bw1, bb1, bw2, bb2,
                tw0, tb0, tw1, tb1, tw2, tb2, tw3, tb3, tw4, tb4,
                out_ref, acc_ref):
    h = dense_ref[:]
    h = jnp.maximum(h @ bw0[:] + bb0[:], 0.0)
    h = jnp.maximum(h @ bw1[:] + bb1[:], 0.0)
    bot = jnp.maximum(h @ bw2[:] + bb2[:], 0.0)

    feat = jnp.concatenate([bot.reshape(_BLK, 1, _EMBED), embed_ref[:]], axis=1)
    xact = jax.lax.dot_general(
        feat, feat, (((2,), (2,)), ((0,), (0,))),
        preferred_element_type=jnp.float32)
    acc_ref[:, 0:_EMBED] = bot
    off = _EMBED
    for i in range(_N_FEAT):
        w = _N_FEAT - i
        acc_ref[:, off:off + w] = xact[:, i, i:]
        off += w

    x = acc_ref[:]
    x = jnp.maximum(x @ tw0[:] + tb0[:], 0.0)
    x = jnp.maximum(x @ tw1[:] + tb1[:], 0.0)
    x = jnp.maximum(x @ tw2[:] + tb2[:], 0.0)
    x = jnp.maximum(x @ tw3[:] + tb3[:], 0.0)
    x = x @ tw4[:] + tb4[:]
    out_ref[:] = jax.nn.sigmoid(x)


def _dense_call(dense, embed, bw0, bb0, bw1, bb1, bw2, bb2,
                tw0, tb0, tw1, tb1, tw2, tb2, tw3, tb3, tw4, tb4):
    def full(a):
        return pl.BlockSpec(a.shape, lambda i: (0,) * a.ndim)

    ws = (bw0, bb0, bw1, bb1, bw2, bb2,
          tw0, tb0, tw1, tb1, tw2, tb2, tw3, tb3, tw4, tb4)
    return pl.pallas_call(
        _dense_body,
        grid=(_GRID,),
        in_specs=[
            pl.BlockSpec((_BLK, dense.shape[1]), lambda i: (i, 0)),
            pl.BlockSpec((_BLK, _N_TABLES, _EMBED), lambda i: (i, 0, 0)),
        ] + [full(w) for w in ws],
        out_specs=pl.BlockSpec((_BLK, 1), lambda i: (i, 0)),
        out_shape=jax.ShapeDtypeStruct((_B, 1), jnp.float32),
        scratch_shapes=[pltpu.VMEM((_BLK, _EMBED + _DI_DIM), jnp.float32)],
        compiler_params=pltpu.CompilerParams(
            dimension_semantics=("arbitrary",),
        ),
    )(dense, embed, *ws)


def kernel(dense_features, cat_features, embedding_table,
           bw0, bb0, bw1, bb1, bw2, bb2,
           tw0, tb0, tw1, tb1, tw2, tb2, tw3, tb3, tw4, tb4):
    offsets = jnp.asarray(np.arange(_N_TABLES, dtype=np.int32) * _VOCAB)
    idx = (cat_features + offsets[None, :]).reshape(_NW, _NCHUNK, _CHUNK)
    rows = _sc_gather(embedding_table, idx)
    embed = rows.reshape(_B, _N_TABLES, _EMBED)
    b2 = lambda v: v.reshape(1, -1)
    return _dense_call(dense_features, embed,
                       bw0, b2(bb0), bw1, b2(bb1), bw2, b2(bb2),
                       tw0, b2(tb0), tw1, b2(tb1), tw2, b2(tb2),
                       tw3, b2(tb3), tw4, b2(tb4))


# speedup vs baseline: 1.4628x; 1.0103x over previous
import functools

import numpy as np
import jax
import jax.numpy as jnp
from jax import lax
from jax.experimental import pallas as pl
from jax.experimental.pallas import tpu as pltpu
from jax.experimental.pallas import tpu_sc as plsc

_VOCAB = 100000
_N_TABLES = 26
_EMBED = 32
_B = 4096
_N_FEAT = 1 + _N_TABLES
_DI_DIM = _N_FEAT * (_N_FEAT + 1) // 2


_NC, _NS = 2, 16
_NW = _NC * _NS
_TOTAL = _B * _N_TABLES
_BPW = _TOTAL // _NW
_CHUNK = 128
_NCHUNK = _BPW // _CHUNK


def _sc_gather(table_pad, idx3d):
    mesh = plsc.VectorSubcoreMesh(core_axis_name="c", subcore_axis_name="s")

    @functools.partial(
        pl.kernel,
        mesh=mesh,
        out_type=jax.ShapeDtypeStruct((_TOTAL, 128), jnp.float32),
        scratch_types=[
            pltpu.VMEM((_NCHUNK, _CHUNK), jnp.int32),
            pltpu.VMEM((_CHUNK, 128), jnp.float32),
            pltpu.VMEM((_CHUNK, 128), jnp.float32),
            pltpu.SemaphoreType.DMA,
        ],
        compiler_params=pltpu.CompilerParams(needs_layout_passes=False),
    )
    def k(table_hbm, idx_hbm, out_hbm, idx_v, g0, g1, sem):
        wid = lax.axis_index("s") * _NC + lax.axis_index("c")
        base = wid * _BPW
        pltpu.sync_copy(idx_hbm.at[wid], idx_v)

        def fire(c, gbuf):
            pltpu.async_copy(table_hbm.at[idx_v.at[c]], gbuf, sem)

        def drain(gbuf):
            pltpu.make_async_copy(
                table_hbm.at[pl.ds(0, _CHUNK)], gbuf, sem).wait()

        def flush(c, gbuf):
            pltpu.sync_copy(
                gbuf, out_hbm.at[pl.ds(base + c * _CHUNK, _CHUNK)])

        fire(0, g0)

        def body(c2, carry):
            c = 2 * c2

            @pl.when(c + 1 < _NCHUNK)
            def _():
                fire(c + 1, g1)

            drain(g0)
            flush(c, g0)

            @pl.when(c + 2 < _NCHUNK)
            def _():
                fire(c + 2, g0)

            @pl.when(c + 1 < _NCHUNK)
            def _():
                drain(g1)
                flush(c + 1, g1)
            return carry

        lax.fori_loop(0, (_NCHUNK + 1) // 2, body, 0)

    return k(table_pad, idx3d)


_BLK = 256
_GRID = _B // _BLK


def _dense_body(dense_ref, embed_ref,
                bw0, bb0, bw1, bb1, bw2, bb2,
                tw0, tb0, tw1, tb1, tw2, tb2, tw3, tb3, tw4, tb4,
                out_ref, acc_ref):
    h = dense_ref[:]
    h = jnp.maximum(h @ bw0[:] + bb0[:], 0.0)
    h = jnp.maximum(h @ bw1[:] + bb1[:], 0.0)
    bot = jnp.maximum(h @ bw2[:] + bb2[:], 0.0)

    feat = jnp.concatenate(
        [bot.reshape(_BLK, 1, _EMBED), embed_ref[:, :, 0:_EMBED]], axis=1)
    xact = jax.lax.dot_general(
        feat, feat, (((2,), (2,)), ((0,), (0,))),
        preferred_element_type=jnp.float32)
    acc_ref[:, 0:_EMBED] = bot
    off = _EMBED
    for i in range(_N_FEAT):
        w = _N_FEAT - i
        acc_ref[:, off:off + w] = xact[:, i, i:]
        off += w

    x = acc_ref[:]
    x = jnp.maximum(x @ tw0[:] + tb0[:], 0.0)
    x = jnp.maximum(x @ tw1[:] + tb1[:], 0.0)
    x = jnp.maximum(x @ tw2[:] + tb2[:], 0.0)
    x = jnp.maximum(x @ tw3[:] + tb3[:], 0.0)
    x = x @ tw4[:] + tb4[:]
    out_ref[:] = jax.nn.sigmoid(x)


def _dense_call(dense, embed, bw0, bb0, bw1, bb1, bw2, bb2,
                tw0, tb0, tw1, tb1, tw2, tb2, tw3, tb3, tw4, tb4):
    def full(a):
        return pl.BlockSpec(a.shape, lambda i: (0,) * a.ndim)

    ws = (bw0, bb0, bw1, bb1, bw2, bb2,
          tw0, tb0, tw1, tb1, tw2, tb2, tw3, tb3, tw4, tb4)
    return pl.pallas_call(
        _dense_body,
        grid=(_GRID,),
        in_specs=[
            pl.BlockSpec((_BLK, dense.shape[1]), lambda i: (i, 0)),
            pl.BlockSpec((_BLK, _N_TABLES, 128), lambda i: (i, 0, 0)),
        ] + [full(w) for w in ws],
        out_specs=pl.BlockSpec((_BLK, 1), lambda i: (i, 0)),
        out_shape=jax.ShapeDtypeStruct((_B, 1), jnp.float32),
        scratch_shapes=[pltpu.VMEM((_BLK, _EMBED + _DI_DIM), jnp.float32)],
        compiler_params=pltpu.CompilerParams(
            dimension_semantics=("arbitrary",),
        ),
    )(dense, embed, *ws)


def kernel(dense_features, cat_features, embedding_table,
           bw0, bb0, bw1, bb1, bw2, bb2,
           tw0, tb0, tw1, tb1, tw2, tb2, tw3, tb3, tw4, tb4):
    offsets = jnp.asarray(np.arange(_N_TABLES, dtype=np.int32) * _VOCAB)
    idx = (cat_features + offsets[None, :]).reshape(_NW, _NCHUNK, _CHUNK)
    table_pad = jnp.pad(embedding_table, ((0, 0), (0, 128 - _EMBED)))
    rows = _sc_gather(table_pad, idx)
    embed = rows.reshape(_B, _N_TABLES, 128)
    b2 = lambda v: v.reshape(1, -1)
    return _dense_call(dense_features, embed,
                       bw0, b2(bb0), bw1, b2(bb1), bw2, b2(bb2),
                       tw0, b2(tb0), tw1, b2(tb1), tw2, b2(tb2),
                       tw3, b2(tb3), tw4, b2(tb4))
